# Initial kernel scaffold; baseline (speedup 1.0000x reference)
import jax
import jax.numpy as jnp
from jax.experimental import pallas as pl


def kernel(edge_index, mc_embeddings, W1, b1, W2, b2):
    raise NotImplementedError("write your pallas kernel here")


# speedup vs baseline: 2.9470x; 2.9470x over previous
import functools

import jax
import jax.numpy as jnp
from jax import lax
from jax.experimental import pallas as pl
from jax.experimental.pallas import tpu as pltpu
from jax.experimental.pallas import tpu_sc as plsc

IN_DIM = 128
HIDDEN_DIM = 128
OUT_DIM = 128
N_NODES = 10000
N_EDGES = 320000
N_CH = 4

SC_CORES = 2
SC_SUBCORES = 16
NW = SC_CORES * SC_SUBCORES
EPW = N_EDGES // NW
CHUNK = 80
N_CHUNKS = EPW // CHUNK


_PRE_BN = 2000


def _pre_body(emb_ref, w1s_ref, w1d_ref, b1_ref, a_ref, b_ref):
    emb = emb_ref[0]
    a_ref[0] = (
        jnp.dot(emb, w1s_ref[...], preferred_element_type=jnp.float32)
        + b1_ref[...]
    )
    b_ref[0] = jnp.dot(emb, w1d_ref[...], preferred_element_type=jnp.float32)


def _precompute(mc_embeddings, w1s_t, w1d_t, b1_row):
    grid = (N_CH, N_NODES // _PRE_BN)
    return pl.pallas_call(
        _pre_body,
        grid=grid,
        in_specs=[
            pl.BlockSpec((1, _PRE_BN, IN_DIM), lambda c, n: (c, n, 0)),
            pl.BlockSpec((IN_DIM, HIDDEN_DIM), lambda c, n: (0, 0)),
            pl.BlockSpec((IN_DIM, HIDDEN_DIM), lambda c, n: (0, 0)),
            pl.BlockSpec((1, HIDDEN_DIM), lambda c, n: (0, 0)),
        ],
        out_specs=[
            pl.BlockSpec((1, _PRE_BN, HIDDEN_DIM), lambda c, n: (c, n, 0)),
            pl.BlockSpec((1, _PRE_BN, HIDDEN_DIM), lambda c, n: (c, n, 0)),
        ],
        out_shape=[
            jax.ShapeDtypeStruct((N_CH, N_NODES, HIDDEN_DIM), jnp.float32),
            jax.ShapeDtypeStruct((N_CH, N_NODES, HIDDEN_DIM), jnp.float32),
        ],
    )(mc_embeddings, w1s_t, w1d_t, b1_row)


def _sc_body(a_hbm, b_hbm, src_hbm, dst_hbm, p0, p1, p2, p3,
             idx_s, idx_d, rows_a, rows_b, sem_a, sem_b):
    cid = lax.axis_index("c")
    sid = lax.axis_index("s")
    wid = sid * SC_CORES + cid
    base = wid * EPW
    outs = (p0, p1, p2, p3)

    def chunk_body(j, carry):
        off = base + j * CHUNK
        pltpu.sync_copy(src_hbm.at[pl.ds(off, CHUNK)], idx_s)
        pltpu.sync_copy(dst_hbm.at[pl.ds(off, CHUNK)], idx_d)
        for ch in range(N_CH):
            ga = pltpu.async_copy(a_hbm.at[ch].at[idx_s], rows_a, sem_a)
            gb = pltpu.async_copy(b_hbm.at[ch].at[idx_d], rows_b, sem_b)
            ga.wait()
            gb.wait()

            def add_body(r, c2):
                for c in range(HIDDEN_DIM // 16):
                    va = rows_a[r, pl.ds(c * 16, 16)]
                    vb = rows_b[r, pl.ds(c * 16, 16)]
                    rows_a[r, pl.ds(c * 16, 16)] = jnp.maximum(va + vb, 0.0)
                return c2

            lax.fori_loop(0, CHUNK, add_body, 0, unroll=False)
            pltpu.sync_copy(rows_a, outs[ch].at[pl.ds(off, CHUNK)])
        return carry

    lax.fori_loop(0, N_CHUNKS, chunk_body, 0, unroll=False)


def _sc_gather(a_tab, b_tab, src, dst):
    mesh = plsc.VectorSubcoreMesh(
        core_axis_name="c", subcore_axis_name="s",
        num_cores=SC_CORES, num_subcores=SC_SUBCORES,
    )
    out_t = [jax.ShapeDtypeStruct((N_EDGES, HIDDEN_DIM), jnp.float32)] * N_CH
    f = pl.kernel(
        _sc_body,
        out_type=out_t,
        mesh=mesh,
        scratch_types=[
            pltpu.VMEM((CHUNK,), jnp.int32),
            pltpu.VMEM((CHUNK,), jnp.int32),
            pltpu.VMEM((CHUNK, HIDDEN_DIM), jnp.float32),
            pltpu.VMEM((CHUNK, HIDDEN_DIM), jnp.float32),
            pltpu.SemaphoreType.DMA,
            pltpu.SemaphoreType.DMA,
        ],
    )
    return f(a_tab, b_tab, src, dst)


_MM_BE = 2000


def _mm_body(p0, p1, p2, p3, w2t_ref, b2_ref, o0, o1, o2, o3):
    w2t = w2t_ref[...]
    b2v = b2_ref[...]
    for p_ref, o_ref in ((p0, o0), (p1, o1), (p2, o2), (p3, o3)):
        o_ref[...] = (
            jnp.dot(p_ref[...], w2t, preferred_element_type=jnp.float32) + b2v
        )


def _final_mm(p_list, w2_t, b2_row):
    grid = (N_EDGES // _MM_BE,)
    io_spec = pl.BlockSpec((_MM_BE, HIDDEN_DIM), lambda e: (e, 0))
    return pl.pallas_call(
        _mm_body,
        grid=grid,
        in_specs=[io_spec] * N_CH + [
            pl.BlockSpec((HIDDEN_DIM, OUT_DIM), lambda e: (0, 0)),
            pl.BlockSpec((1, OUT_DIM), lambda e: (0, 0)),
        ],
        out_specs=[pl.BlockSpec((_MM_BE, OUT_DIM), lambda e: (e, 0))] * N_CH,
        out_shape=[jax.ShapeDtypeStruct((N_EDGES, OUT_DIM), jnp.float32)] * N_CH,
    )(*p_list, w2_t, b2_row)


def kernel(edge_index, mc_embeddings, W1, b1, W2, b2):
    w1s_t = W1[:, :IN_DIM].T
    w1d_t = W1[:, IN_DIM:].T
    w2_t = W2.T
    a_tab, b_tab = _precompute(
        mc_embeddings, w1s_t, w1d_t, b1.reshape(1, HIDDEN_DIM)
    )
    p_list = _sc_gather(a_tab, b_tab, edge_index[0], edge_index[1])
    outs = _final_mm(p_list, w2_t, b2.reshape(1, OUT_DIM))
    return tuple(outs)


# speedup vs baseline: 4.5117x; 1.5309x over previous
import functools

import jax
import jax.numpy as jnp
from jax import lax
from jax.experimental import pallas as pl
from jax.experimental.pallas import tpu as pltpu
from jax.experimental.pallas import tpu_sc as plsc

IN_DIM = 128
HIDDEN_DIM = 128
OUT_DIM = 128
N_NODES = 10000
N_EDGES = 320000
N_CH = 4

SC_CORES = 2
SC_SUBCORES = 16
NW = SC_CORES * SC_SUBCORES
EPW = N_EDGES // NW
CHUNK = 80
N_CHUNKS = EPW // CHUNK


_PRE_BN = 2000


def _pre_body(emb_ref, w1s_ref, w1d_ref, b1_ref, a_ref, b_ref):
    emb = emb_ref[0]
    a_ref[0] = (
        jnp.dot(emb, w1s_ref[...], preferred_element_type=jnp.float32)
        + b1_ref[...]
    )
    b_ref[0] = jnp.dot(emb, w1d_ref[...], preferred_element_type=jnp.float32)


def _precompute(mc_embeddings, w1s_t, w1d_t, b1_row):
    grid = (N_CH, N_NODES // _PRE_BN)
    return pl.pallas_call(
        _pre_body,
        grid=grid,
        in_specs=[
            pl.BlockSpec((1, _PRE_BN, IN_DIM), lambda c, n: (c, n, 0)),
            pl.BlockSpec((IN_DIM, HIDDEN_DIM), lambda c, n: (0, 0)),
            pl.BlockSpec((IN_DIM, HIDDEN_DIM), lambda c, n: (0, 0)),
            pl.BlockSpec((1, HIDDEN_DIM), lambda c, n: (0, 0)),
        ],
        out_specs=[
            pl.BlockSpec((1, _PRE_BN, HIDDEN_DIM), lambda c, n: (c, n, 0)),
            pl.BlockSpec((1, _PRE_BN, HIDDEN_DIM), lambda c, n: (c, n, 0)),
        ],
        out_shape=[
            jax.ShapeDtypeStruct((N_CH, N_NODES, HIDDEN_DIM), jnp.float32),
            jax.ShapeDtypeStruct((N_CH, N_NODES, HIDDEN_DIM), jnp.float32),
        ],
    )(mc_embeddings, w1s_t, w1d_t, b1_row)


_MAIN_PAIRS = (N_CHUNKS - 1) // 2


def _sc_body(a_hbm, b_hbm, src_hbm, dst_hbm, p0, p1, p2, p3,
             idx_s, idx_d, buf_a0, buf_a1, buf_b0, buf_b1,
             sem_g0, sem_g1, sem_s0, sem_s1):
    cid = lax.axis_index("c")
    sid = lax.axis_index("s")
    wid = sid * SC_CORES + cid
    base = wid * EPW
    outs = (p0, p1, p2, p3)
    buf_a = (buf_a0, buf_a1)
    buf_b = (buf_b0, buf_b1)
    sem_g = (sem_g0, sem_g1)
    sem_s = (sem_s0, sem_s1)

    pltpu.sync_copy(src_hbm.at[pl.ds(base, EPW)], idx_s)
    pltpu.sync_copy(dst_hbm.at[pl.ds(base, EPW)], idx_d)

    def issue_gathers(j, ch, p):
        isl = idx_s.at[pl.ds(j * CHUNK, CHUNK)]
        idl = idx_d.at[pl.ds(j * CHUNK, CHUNK)]
        pltpu.async_copy(a_hbm.at[ch].at[isl], buf_a[p], sem_g[p])
        pltpu.async_copy(b_hbm.at[ch].at[idl], buf_b[p], sem_g[p])

    def wait_gathers(p):
        isl = idx_s.at[pl.ds(0, CHUNK)]
        idl = idx_d.at[pl.ds(0, CHUNK)]
        pltpu.make_async_copy(a_hbm.at[0].at[isl], buf_a[p], sem_g[p]).wait()
        pltpu.make_async_copy(b_hbm.at[0].at[idl], buf_b[p], sem_g[p]).wait()

    def wait_scatter(p):
        pltpu.make_async_copy(
            buf_a[p], outs[0].at[pl.ds(base, CHUNK)], sem_s[p]
        ).wait()

    def step(j, ch, jn, chn, guard_j2=None, last=False):
        p = ch & 1
        q = 1 - p
        if not last:
            if guard_j2 is None:
                wait_scatter(q)
            else:
                @pl.when(guard_j2 > 0)
                def _():
                    wait_scatter(q)
            issue_gathers(jn, chn, q)
        wait_gathers(p)
        a = buf_a[p]
        b = buf_b[p]

        def add_body(r, carry):
            for c in range(HIDDEN_DIM // 16):
                plsc.addupdate(
                    a.at[r, pl.ds(c * 16, 16)], b[r, pl.ds(c * 16, 16)]
                )
            return carry

        lax.fori_loop(0, CHUNK, add_body, 0)
        pltpu.async_copy(
            a, outs[ch].at[pl.ds(base + j * CHUNK, CHUNK)], sem_s[p]
        )

    issue_gathers(0, 0, 0)

    def body2(j2, carry):
        for jp in range(2):
            j = 2 * j2 + jp
            for ch in range(N_CH):
                chn = (ch + 1) % N_CH
                jn = j + (1 if ch == N_CH - 1 else 0)
                guard = j2 if (jp == 0 and ch == 0) else None
                step(j, ch, jn, chn, guard_j2=guard)
        return carry

    lax.fori_loop(0, _MAIN_PAIRS, body2, 0)

    j_tail = N_CHUNKS - 1
    for ch in range(N_CH):
        chn = (ch + 1) % N_CH
        step(j_tail, ch, j_tail, chn, last=(ch == N_CH - 1))

    wait_scatter(0)
    wait_scatter(1)


def _sc_gather(a_tab, b_tab, src, dst):
    mesh = plsc.VectorSubcoreMesh(
        core_axis_name="c", subcore_axis_name="s",
        num_cores=SC_CORES, num_subcores=SC_SUBCORES,
    )
    out_t = [jax.ShapeDtypeStruct((N_EDGES, HIDDEN_DIM), jnp.float32)] * N_CH
    f = pl.kernel(
        _sc_body,
        out_type=out_t,
        mesh=mesh,
        scratch_types=[
            pltpu.VMEM((EPW,), jnp.int32),
            pltpu.VMEM((EPW,), jnp.int32),
            pltpu.VMEM((CHUNK, HIDDEN_DIM), jnp.float32),
            pltpu.VMEM((CHUNK, HIDDEN_DIM), jnp.float32),
            pltpu.VMEM((CHUNK, HIDDEN_DIM), jnp.float32),
            pltpu.VMEM((CHUNK, HIDDEN_DIM), jnp.float32),
            pltpu.SemaphoreType.DMA,
            pltpu.SemaphoreType.DMA,
            pltpu.SemaphoreType.DMA,
            pltpu.SemaphoreType.DMA,
        ],
    )
    return f(a_tab, b_tab, src, dst)


_MM_BE = 2000


def _mm_body(p0, p1, p2, p3, w2t_ref, b2_ref, o0, o1, o2, o3):
    w2t = w2t_ref[...]
    b2v = b2_ref[...]
    for p_ref, o_ref in ((p0, o0), (p1, o1), (p2, o2), (p3, o3)):
        h = jnp.maximum(p_ref[...], 0.0)
        o_ref[...] = jnp.dot(h, w2t, preferred_element_type=jnp.float32) + b2v


def _final_mm(p_list, w2_t, b2_row):
    grid = (N_EDGES // _MM_BE,)
    io_spec = pl.BlockSpec((_MM_BE, HIDDEN_DIM), lambda e: (e, 0))
    return pl.pallas_call(
        _mm_body,
        grid=grid,
        in_specs=[io_spec] * N_CH + [
            pl.BlockSpec((HIDDEN_DIM, OUT_DIM), lambda e: (0, 0)),
            pl.BlockSpec((1, OUT_DIM), lambda e: (0, 0)),
        ],
        out_specs=[pl.BlockSpec((_MM_BE, OUT_DIM), lambda e: (e, 0))] * N_CH,
        out_shape=[jax.ShapeDtypeStruct((N_EDGES, OUT_DIM), jnp.float32)] * N_CH,
    )(*p_list, w2_t, b2_row)


def kernel(edge_index, mc_embeddings, W1, b1, W2, b2):
    w1s_t = W1[:, :IN_DIM].T
    w1d_t = W1[:, IN_DIM:].T
    w2_t = W2.T
    a_tab, b_tab = _precompute(
        mc_embeddings, w1s_t, w1d_t, b1.reshape(1, HIDDEN_DIM)
    )
    p_list = _sc_gather(a_tab, b_tab, edge_index[0], edge_index[1])
    outs = _final_mm(p_list, w2_t, b2.reshape(1, OUT_DIM))
    return tuple(outs)
